# baseline (speedup 1.0000x reference)
import jax
import jax.numpy as jnp
from jax.experimental import pallas as pl
from jax.experimental.pallas import tpu as pltpu

_N = 4096
_D = 512
_NUM_CLASSES = 64
_MARGIN = 0.5
_BLK = 256
_KPAD = 640
_TWO_B = 16384.0


def _triplet_kernel(a_ref, e_ref, out_ref):
    i = pl.program_id(0)

    s = jax.lax.dot_general(
        a_ref[...], e_ref[...], (((1,), (1,)), ((), ())),
        preferred_element_type=jnp.float32,
    )

    pos_red = jnp.max(s, axis=1, keepdims=True) - _TWO_B
    neg_red = jnp.min(s, axis=1, keepdims=True)

    blk_loss = jnp.sum(
        jnp.maximum(pos_red - neg_red + _MARGIN, 0.0), keepdims=True
    ).reshape(1, 1)

    @pl.when(i == 0)
    def _init():
        out_ref[...] = jnp.zeros_like(out_ref)

    out_ref[...] += blk_loss


def kernel(embeds, labels):
    f = jnp.float32
    sq = jnp.sum(embeds * embeds, axis=1, keepdims=True)
    sq_hi = sq.astype(jnp.bfloat16).astype(f)
    sq_lo = sq - sq_hi
    onehot = (
        labels[:, None] == jnp.arange(_NUM_CLASSES, dtype=labels.dtype)[None, :]
    ).astype(f) * 128.0

    ones = jnp.ones((_N, 1), f)
    zpad = jnp.zeros((_N, _KPAD - _D - 2 - _NUM_CLASSES), f)
    a_aug = jnp.concatenate([-2.0 * embeds, ones, ones, onehot, zpad], axis=1)
    e_aug = jnp.concatenate([embeds, sq_hi, sq_lo, onehot, zpad], axis=1)
    a_aug = a_aug.astype(jnp.bfloat16)
    e_aug = e_aug.astype(jnp.bfloat16)

    total = pl.pallas_call(
        _triplet_kernel,
        grid=(_N // _BLK,),
        in_specs=[
            pl.BlockSpec((_BLK, _KPAD), lambda i: (i, 0)),
            pl.BlockSpec((_N, _KPAD), lambda i: (0, 0)),
        ],
        out_specs=pl.BlockSpec((1, 1), lambda i: (0, 0)),
        out_shape=jax.ShapeDtypeStruct((1, 1), jnp.float32),
        compiler_params=pltpu.CompilerParams(
            dimension_semantics=("arbitrary",),
        ),
    )(a_aug, e_aug)

    return total[0, 0] / _N


# speedup vs baseline: 1.7281x; 1.7281x over previous
import jax
import jax.numpy as jnp
from jax.experimental import pallas as pl
from jax.experimental.pallas import tpu as pltpu

_N = 4096
_D = 512
_NUM_CLASSES = 64
_MARGIN = 0.5
_BLK = 256
_KAUG = 128
_TWO_B = 16384.0


def _triplet_kernel(e_ref, lab_ref, out_ref, e16_s, lhs_s, rhs_s):
    i = pl.program_id(0)
    f = jnp.float32

    @pl.when(i == 0)
    def _prologue():
        e = e_ref[...]
        e16_s[...] = e.astype(jnp.bfloat16)
        oh = jnp.where(
            lab_ref[...] == jax.lax.broadcasted_iota(
                jnp.int32, (_N, _NUM_CLASSES), 1),
            jnp.float32(128.0), jnp.float32(0.0))
        sq = jnp.sum(e * e, axis=1, keepdims=True)
        hi = sq.astype(jnp.bfloat16).astype(f)
        lo = sq - hi
        ones = jnp.ones((_N, 1), f)
        zpad = jnp.zeros((_N, _KAUG - 2 - _NUM_CLASSES), f)
        lhs_s[...] = jnp.concatenate(
            [ones, ones, oh, zpad], axis=1).astype(jnp.bfloat16)
        rhs_s[...] = jnp.concatenate(
            [hi, lo, oh, zpad], axis=1).astype(jnp.bfloat16)
        out_ref[...] = jnp.zeros_like(out_ref)

    a2 = (-2.0 * e_ref[pl.ds(i * _BLK, _BLK), :]).astype(jnp.bfloat16)
    dims = (((1,), (1,)), ((), ()))
    s = jax.lax.dot_general(a2, e16_s[...], dims, preferred_element_type=f)
    s += jax.lax.dot_general(
        lhs_s[pl.ds(i * _BLK, _BLK), :], rhs_s[...], dims,
        preferred_element_type=f)

    pos_red = jnp.max(s, axis=1, keepdims=True) - _TWO_B
    neg_red = jnp.min(s, axis=1, keepdims=True)

    out_ref[...] += jnp.sum(
        jnp.maximum(pos_red - neg_red + _MARGIN, 0.0), keepdims=True
    ).reshape(1, 1)


def kernel(embeds, labels):
    total = pl.pallas_call(
        _triplet_kernel,
        grid=(_N // _BLK,),
        in_specs=[
            pl.BlockSpec((_N, _D), lambda i: (0, 0)),
            pl.BlockSpec((_N, 1), lambda i: (0, 0)),
        ],
        out_specs=pl.BlockSpec((1, 1), lambda i: (0, 0)),
        out_shape=jax.ShapeDtypeStruct((1, 1), jnp.float32),
        scratch_shapes=[
            pltpu.VMEM((_N, _D), jnp.bfloat16),
            pltpu.VMEM((_N, _KAUG), jnp.bfloat16),
            pltpu.VMEM((_N, _KAUG), jnp.bfloat16),
        ],
        compiler_params=pltpu.CompilerParams(
            dimension_semantics=("arbitrary",),
        ),
    )(embeds, labels.reshape(_N, 1))

    return total[0, 0] / _N
